# baseline (speedup 1.0000x reference)
import jax
import jax.numpy as jnp
from jax import lax
from jax.experimental import pallas as pl
from jax.experimental.pallas import tpu as pltpu
from jax.experimental.pallas import tpu_sc as plsc

B, T = 64, 128
N = B * T
EMB, HID, K, CONT, V0, V1, CED = 256, 128, 8192, 16, 32, 64, 6

MT = 512
NT = N // MT
RT = MT // T
KC = 2048
KP = 1024
MD = 1024
ND = N // MD
GW = 128


def _dot(a, b, dims=None):
    if dims is None:
        dims = (((1,), (0,)), ((), ()))
    return lax.dot_general(a, b, dims, preferred_element_type=jnp.float32)


def _nf_body(naipf_ref, convwf_ref, convb_ref, npw_ref, npb_ref, o_ref):
    naipf = jnp.nan_to_num(naipf_ref[...], nan=0.0, posinf=0.0, neginf=0.0)
    feat = _dot(naipf, convwf_ref[...], (((1,), (1,)), ((), ())))
    feat = jnp.maximum(feat + convb_ref[...], 0.0)
    o_ref[...] = _dot(feat, npw_ref[...]) + npb_ref[...]


def _cbb_body(cb_ref, w_ref, b_ref, ones_ref, cbb_ref, c2_ref):
    cb = cb_ref[...]
    cbb_ref[...] = _dot(cb, w_ref[...]) + b_ref[...]
    c2_ref[...] = _dot(ones_ref[...], cb * cb, (((1,), (1,)), ((), ())))


def _vq_body(cont_ref, cat_ref, nf_ref, emb0_ref, emb1_ref, cpw_ref, cpb_ref,
             kw_ref, kb_ref, f1w_ref, f1b_ref, f2w_ref, f2b_ref,
             cb_ref, c2_ref, idx_ref, dl_ref):
    cont = jnp.nan_to_num(cont_ref[...], nan=0.0, posinf=0.0, neginf=0.0)
    cont_h = _dot(cont, cpw_ref[...]) + cpb_ref[...]

    cat = cat_ref[...]
    oh0 = (cat[:, 0:1] == lax.broadcasted_iota(jnp.int32, (MT, V0), 1)
           ).astype(jnp.float32)
    oh1 = (cat[:, 1:2] == lax.broadcasted_iota(jnp.int32, (MT, V1), 1)
           ).astype(jnp.float32)
    e0 = _dot(oh0, emb0_ref[...])
    e1 = _dot(oh1, emb1_ref[...])
    cat_h = _dot(jnp.concatenate([e0, e1], axis=1), kw_ref[...]) + kb_ref[...]

    na = jnp.broadcast_to(nf_ref[...], (RT, T, HID)).reshape(MT, HID)

    fused = jnp.concatenate([na, cont_h, cat_h], axis=1)
    h1 = jnp.maximum(_dot(fused, f1w_ref[...]) + f1b_ref[...], 0.0)
    z = _dot(h1, f2w_ref[...]) + f2b_ref[...]
    z2 = jnp.sum(z * z, axis=1, keepdims=True)

    run_min = jnp.full((MT, 1), jnp.inf, jnp.float32)
    run_arg = jnp.zeros((MT, 1), jnp.int32)
    for c in range(K // KC):
        cbc = cb_ref[pl.ds(c * KC, KC), :]
        scores = _dot(z, cbc, (((1,), (1,)), ((), ())))
        d = (z2 + c2_ref[0:1, pl.ds(c * KC, KC)]) - 2.0 * scores
        cmin = jnp.min(d, axis=1, keepdims=True)
        lane = lax.broadcasted_iota(jnp.int32, (MT, KC), 1)
        carg = jnp.min(jnp.where(d == cmin, lane, K), axis=1,
                       keepdims=True) + c * KC
        upd = cmin < run_min
        run_min = jnp.where(upd, cmin, run_min)
        run_arg = jnp.where(upd, carg, run_arg)
    idx_ref[0] = run_arg
    dl_ref[...] = jnp.sum(run_min).reshape(1, 1, 1)


def _stat_body(idx_ref, dl_ref, perp_ref, loss_ref, acc_ref):
    i = pl.program_id(0)

    @pl.when(i == 0)
    def _():
        acc_ref[0] = 0.0

    ids = i * 128 + lax.broadcasted_iota(jnp.int32, (1, 128), 1)
    mask = (idx_ref[...] == ids).astype(jnp.float32)
    counts = jnp.sum(mask, axis=0, keepdims=True)
    p = counts / N
    acc_ref[0] += jnp.sum(p * jnp.log(p + 1e-12))

    @pl.when(i == K // 128 - 1)
    def _():
        perp_ref[...] = jnp.exp(-acc_ref[0]).reshape(1, 1)
        loss_ref[...] = (jnp.sum(dl_ref[...]) * (1.25 / (N * EMB))
                         ).reshape(1, 1)


def _dec_body(g_ref, b2w_ref, b2b_ref, chw_ref, chb_ref, c0w_ref, c0b_ref,
              c1w_ref, c1b_ref, sel_ref, co_ref, k0_ref, k1_ref, hl_ref):
    g = jnp.maximum(g_ref[...], 0.0)
    h = jnp.maximum(_dot(g, b2w_ref[...]) + b2b_ref[...], 0.0)
    co_ref[...] = _dot(h, chw_ref[...]) + chb_ref[...]
    k0_ref[...] = _dot(h, c0w_ref[...]) + c0b_ref[...]
    k1_ref[...] = _dot(h, c1w_ref[...]) + c1b_ref[...]
    hl_ref[...] = _dot(sel_ref[...], h)


def _can_body(hl_ref, w1_ref, b1_ref, w2_ref, b2_ref, o_ref):
    r = jnp.maximum(_dot(hl_ref[...], w1_ref[...]) + b1_ref[...], 0.0)
    o_ref[...] = _dot(r, w2_ref[...]) + b2_ref[...]


def _sc_gather(cbb, idx_row):
    mesh = plsc.VectorSubcoreMesh(core_axis_name="core",
                                  subcore_axis_name="subcore")

    @pl.kernel(out_type=jax.ShapeDtypeStruct((N, HID), jnp.float32), mesh=mesh)
    def kern(x_hbm, i_hbm, o_hbm):
        def body(i_vmem, o_vmem):
            pltpu.sync_copy(x_hbm.at[i_vmem.at[0]], o_vmem)

        pltpu.emit_pipeline(
            body,
            grid=(N // GW,),
            in_specs=[pl.BlockSpec((1, GW), index_map=lambda i: (0, i))],
            out_specs=[pl.BlockSpec((GW, HID), index_map=lambda i: (i, 0))],
            core_axis_name="subcore",
            dimension_semantics=(pltpu.PARALLEL,),
        )(i_hbm, o_hbm)

    return kern(cbb, idx_row)


def kernel(cont, cat, naip, emb0, emb1, conv_w, conv_b, naip_proj_w,
           naip_proj_b, cont_proj_w, cont_proj_b, cat_proj_w, cat_proj_b,
           fuse1_w, fuse1_b, fuse2_w, fuse2_b, codebook, bb1_w, bb1_b,
           bb2_w, bb2_b, cont_head_w, cont_head_b, cat0_w, cat0_b,
           cat1_w, cat1_b, can1_w, can1_b, can2_w, can2_b):
    f32 = jnp.float32
    row = lambda v: v.reshape(1, -1)

    nf = pl.pallas_call(
        _nf_body,
        out_shape=jax.ShapeDtypeStruct((B, HID), f32),
    )(naip.reshape(B, 9), conv_w.reshape(HID, 9), row(conv_b),
      naip_proj_w, row(naip_proj_b))

    cbb, c2 = pl.pallas_call(
        _cbb_body,
        grid=(K // KP,),
        in_specs=[
            pl.BlockSpec((KP, EMB), lambda i: (i, 0)),
            pl.BlockSpec((EMB, HID), lambda i: (0, 0)),
            pl.BlockSpec((1, HID), lambda i: (0, 0)),
            pl.BlockSpec((1, EMB), lambda i: (0, 0)),
        ],
        out_specs=[
            pl.BlockSpec((KP, HID), lambda i: (i, 0)),
            pl.BlockSpec((1, KP), lambda i: (0, i)),
        ],
        out_shape=[
            jax.ShapeDtypeStruct((K, HID), f32),
            jax.ShapeDtypeStruct((1, K), f32),
        ],
        compiler_params=pltpu.CompilerParams(
            dimension_semantics=("parallel",)),
    )(codebook, bb1_w, row(bb1_b), jnp.ones((1, EMB), f32))

    idx3, dloss = pl.pallas_call(
        _vq_body,
        grid=(NT,),
        in_specs=[
            pl.BlockSpec((MT, CONT), lambda i: (i, 0)),
            pl.BlockSpec((MT, 2), lambda i: (i, 0)),
            pl.BlockSpec((RT, 1, HID), lambda i: (i, 0, 0)),
            pl.BlockSpec((V0, CED), lambda i: (0, 0)),
            pl.BlockSpec((V1, CED), lambda i: (0, 0)),
            pl.BlockSpec((CONT, HID), lambda i: (0, 0)),
            pl.BlockSpec((1, HID), lambda i: (0, 0)),
            pl.BlockSpec((2 * CED, HID), lambda i: (0, 0)),
            pl.BlockSpec((1, HID), lambda i: (0, 0)),
            pl.BlockSpec((3 * HID, HID), lambda i: (0, 0)),
            pl.BlockSpec((1, HID), lambda i: (0, 0)),
            pl.BlockSpec((HID, EMB), lambda i: (0, 0)),
            pl.BlockSpec((1, EMB), lambda i: (0, 0)),
            pl.BlockSpec((K, EMB), lambda i: (0, 0)),
            pl.BlockSpec((1, K), lambda i: (0, 0)),
        ],
        out_specs=[
            pl.BlockSpec((1, MT, 1), lambda i: (i, 0, 0)),
            pl.BlockSpec((1, 1, 1), lambda i: (i, 0, 0)),
        ],
        out_shape=[
            jax.ShapeDtypeStruct((NT, MT, 1), jnp.int32),
            jax.ShapeDtypeStruct((NT, 1, 1), f32),
        ],
        compiler_params=pltpu.CompilerParams(
            dimension_semantics=("parallel",)),
    )(cont.reshape(N, CONT), cat.reshape(N, 2), nf.reshape(B, 1, HID),
      emb0, emb1,
      cont_proj_w, row(cont_proj_b), cat_proj_w, row(cat_proj_b),
      fuse1_w, row(fuse1_b), fuse2_w, row(fuse2_b), codebook, c2)

    idx_flat = idx3.reshape(N)

    g = _sc_gather(cbb, idx_flat.reshape(1, N))

    perp, vq_loss = pl.pallas_call(
        _stat_body,
        grid=(K // 128,),
        in_specs=[
            pl.BlockSpec((N, 1), lambda i: (0, 0)),
            pl.BlockSpec((NT, 1, 1), lambda i: (0, 0, 0)),
        ],
        out_specs=[
            pl.BlockSpec((1, 1), lambda i: (0, 0)),
            pl.BlockSpec((1, 1), lambda i: (0, 0)),
        ],
        out_shape=[
            jax.ShapeDtypeStruct((1, 1), f32),
            jax.ShapeDtypeStruct((1, 1), f32),
        ],
        scratch_shapes=[pltpu.SMEM((1,), f32)],
    )(idx_flat.reshape(N, 1), dloss)

    sel = (jnp.arange(MD)[None, :] ==
           (jnp.arange(MD // T) * T + (T - 1))[:, None]).astype(f32)
    cont_out, cat0, cat1, h_last = pl.pallas_call(
        _dec_body,
        grid=(ND,),
        in_specs=[
            pl.BlockSpec((MD, HID), lambda i: (i, 0)),
            pl.BlockSpec((HID, HID), lambda i: (0, 0)),
            pl.BlockSpec((1, HID), lambda i: (0, 0)),
            pl.BlockSpec((HID, CONT), lambda i: (0, 0)),
            pl.BlockSpec((1, CONT), lambda i: (0, 0)),
            pl.BlockSpec((HID, V0), lambda i: (0, 0)),
            pl.BlockSpec((1, V0), lambda i: (0, 0)),
            pl.BlockSpec((HID, V1), lambda i: (0, 0)),
            pl.BlockSpec((1, V1), lambda i: (0, 0)),
            pl.BlockSpec((MD // T, MD), lambda i: (0, 0)),
        ],
        out_specs=[
            pl.BlockSpec((MD, CONT), lambda i: (i, 0)),
            pl.BlockSpec((MD, V0), lambda i: (i, 0)),
            pl.BlockSpec((MD, V1), lambda i: (i, 0)),
            pl.BlockSpec((MD // T, HID), lambda i: (i, 0)),
        ],
        out_shape=[
            jax.ShapeDtypeStruct((N, CONT), f32),
            jax.ShapeDtypeStruct((N, V0), f32),
            jax.ShapeDtypeStruct((N, V1), f32),
            jax.ShapeDtypeStruct((B, HID), f32),
        ],
        compiler_params=pltpu.CompilerParams(
            dimension_semantics=("parallel",)),
    )(g, bb2_w, row(bb2_b), cont_head_w, row(cont_head_b),
      cat0_w, row(cat0_b), cat1_w, row(cat1_b), sel)

    canopy = pl.pallas_call(
        _can_body,
        out_shape=jax.ShapeDtypeStruct((B, 1), f32),
    )(h_last, can1_w, row(can1_b), can2_w, row(can2_b))

    return (cont_out.reshape(B, T, CONT), cat0.reshape(B, T, V0),
            cat1.reshape(B, T, V1), canopy.reshape(B),
            vq_loss.reshape(()), idx_flat.reshape(B, T),
            perp.reshape(()))


# speedup vs baseline: 1.5087x; 1.5087x over previous
import functools

import jax
import jax.numpy as jnp
from jax import lax
from jax.experimental import pallas as pl
from jax.experimental.pallas import tpu as pltpu
from jax.experimental.pallas import tpu_sc as plsc

B, T = 64, 128
N = B * T
EMB, HID, K, CONT, V0, V1, CED = 256, 128, 8192, 16, 32, 64, 6

MT = 512
NT = N // MT
RT = MT // T
KC = 2048
KP = 1024
MD = 1024
ND = N // MD
GW = 128


def _dot(a, b, dims=None):
    if dims is None:
        dims = (((1,), (0,)), ((), ()))
    return lax.dot_general(a, b, dims, precision=lax.Precision.DEFAULT,
                           preferred_element_type=jnp.float32)


def _sel(a, b, dims=None):
    if dims is None:
        dims = (((1,), (0,)), ((), ()))
    return lax.dot_general(a, b, dims, precision=lax.Precision.HIGHEST,
                           preferred_element_type=jnp.float32)


def _nf_body(naipf_ref, convwf_ref, convb_ref, npw_ref, npb_ref, o_ref):
    naipf = jnp.nan_to_num(naipf_ref[...], nan=0.0, posinf=0.0, neginf=0.0)
    feat = _dot(naipf, convwf_ref[...], (((1,), (1,)), ((), ())))
    feat = jnp.maximum(feat + convb_ref[...], 0.0)
    o_ref[...] = _dot(feat, npw_ref[...]) + npb_ref[...]


def _cbb_body(cb_ref, w_ref, b_ref, cbb_ref):
    cbb_ref[...] = _dot(cb_ref[...], w_ref[...]) + b_ref[...]


def _enc_body(cont_ref, cat_ref, nf_ref, emb0_ref, emb1_ref, cpw_ref, cpb_ref,
              kw_ref, kb_ref, f1w_ref, f1b_ref, f2w_ref, f2b_ref, z_ref):
    cont = jnp.nan_to_num(cont_ref[...], nan=0.0, posinf=0.0, neginf=0.0)
    cont_h = _dot(cont, cpw_ref[...]) + cpb_ref[...]

    cat = cat_ref[...]
    oh0 = (cat[:, 0:1] == lax.broadcasted_iota(jnp.int32, (MT, V0), 1)
           ).astype(jnp.float32)
    oh1 = (cat[:, 1:2] == lax.broadcasted_iota(jnp.int32, (MT, V1), 1)
           ).astype(jnp.float32)
    e0 = _sel(oh0, emb0_ref[...])
    e1 = _sel(oh1, emb1_ref[...])
    cat_h = _dot(jnp.concatenate([e0, e1], axis=1), kw_ref[...]) + kb_ref[...]

    na = jnp.broadcast_to(nf_ref[...], (RT, T, HID)).reshape(MT, HID)

    fused = jnp.concatenate([na, cont_h, cat_h], axis=1)
    h1 = jnp.maximum(_dot(fused, f1w_ref[...]) + f1b_ref[...], 0.0)
    z_ref[...] = _dot(h1, f2w_ref[...]) + f2b_ref[...]


def _vq_body(z_ref, z2_ref, c2_ref, cb_ref, idx_ref, dl_ref):
    z = z_ref[...]
    z2 = z2_ref[...]
    run_min = jnp.full((MT, 1), jnp.inf, jnp.float32)
    run_arg = jnp.zeros((MT, 1), jnp.int32)
    for c in range(K // KC):
        cbc = cb_ref[pl.ds(c * KC, KC), :]
        scores = _dot(z, cbc, (((1,), (1,)), ((), ())))
        d = (z2 + c2_ref[0:1, pl.ds(c * KC, KC)]) - 2.0 * scores
        cmin = jnp.min(d, axis=1, keepdims=True)
        lane = lax.broadcasted_iota(jnp.int32, (MT, KC), 1)
        carg = jnp.min(jnp.where(d == cmin, lane, K), axis=1,
                       keepdims=True) + c * KC
        upd = cmin < run_min
        run_min = jnp.where(upd, cmin, run_min)
        run_arg = jnp.where(upd, carg, run_arg)
    idx_ref[0] = run_arg
    dl_ref[...] = jnp.sum(run_min).reshape(1, 1, 1)


def _stat_body(idx_ref, dl_ref, perp_ref, loss_ref, acc_ref):
    i = pl.program_id(0)

    @pl.when(i == 0)
    def _():
        acc_ref[0] = 0.0

    ids = i * 128 + lax.broadcasted_iota(jnp.int32, (1, 128), 1)
    mask = (idx_ref[...] == ids).astype(jnp.float32)
    counts = jnp.sum(mask, axis=0, keepdims=True)
    p = counts / N
    acc_ref[0] += jnp.sum(p * jnp.log(p + 1e-12))

    @pl.when(i == K // 128 - 1)
    def _():
        perp_ref[...] = jnp.exp(-acc_ref[0]).reshape(1, 1)
        loss_ref[...] = (jnp.sum(dl_ref[...]) * (1.25 / (N * EMB))
                         ).reshape(1, 1)


def _dec_body(g_ref, b2w_ref, b2b_ref, chw_ref, chb_ref, c0w_ref, c0b_ref,
              c1w_ref, c1b_ref, sel_ref, co_ref, k0_ref, k1_ref, hl_ref):
    g = jnp.maximum(g_ref[...], 0.0)
    h = jnp.maximum(_dot(g, b2w_ref[...]) + b2b_ref[...], 0.0)
    co_ref[...] = _dot(h, chw_ref[...]) + chb_ref[...]
    k0_ref[...] = _dot(h, c0w_ref[...]) + c0b_ref[...]
    k1_ref[...] = _dot(h, c1w_ref[...]) + c1b_ref[...]
    hl_ref[...] = _sel(sel_ref[...], h)


def _can_body(hl_ref, w1_ref, b1_ref, w2_ref, b2_ref, o_ref):
    r = jnp.maximum(_dot(hl_ref[...], w1_ref[...]) + b1_ref[...], 0.0)
    o_ref[...] = _dot(r, w2_ref[...]) + b2_ref[...]


_NW = 32
_BW = N // _NW


def _sc_gather(cbb, idx_flat):
    mesh = plsc.VectorSubcoreMesh(core_axis_name="c", subcore_axis_name="s")

    @functools.partial(
        pl.kernel, mesh=mesh,
        out_type=jax.ShapeDtypeStruct((N, HID), jnp.float32),
        scratch_types=[
            pltpu.VMEM((_BW,), jnp.int32),
            pltpu.VMEM((_BW, HID), jnp.float32),
            pltpu.SemaphoreType.DMA,
        ],
    )
    def kern(table_hbm, idx_hbm, out_hbm, idx_v, rows_v, sem):
        wid = lax.axis_index("s") * 2 + lax.axis_index("c")
        base = wid * _BW
        pltpu.sync_copy(idx_hbm.at[pl.ds(base, _BW)], idx_v)
        pltpu.async_copy(table_hbm.at[idx_v], rows_v, sem).wait()
        pltpu.sync_copy(rows_v, out_hbm.at[pl.ds(base, _BW)])

    return kern(cbb, idx_flat)


def kernel(cont, cat, naip, emb0, emb1, conv_w, conv_b, naip_proj_w,
           naip_proj_b, cont_proj_w, cont_proj_b, cat_proj_w, cat_proj_b,
           fuse1_w, fuse1_b, fuse2_w, fuse2_b, codebook, bb1_w, bb1_b,
           bb2_w, bb2_b, cont_head_w, cont_head_b, cat0_w, cat0_b,
           cat1_w, cat1_b, can1_w, can1_b, can2_w, can2_b):
    f32 = jnp.float32
    row = lambda v: v.reshape(1, -1)

    nf = pl.pallas_call(
        _nf_body,
        out_shape=jax.ShapeDtypeStruct((B, HID), f32),
    )(naip.reshape(B, 9), conv_w.reshape(HID, 9), row(conv_b),
      naip_proj_w, row(naip_proj_b))

    cbb = pl.pallas_call(
        _cbb_body,
        grid=(K // KP,),
        in_specs=[
            pl.BlockSpec((KP, EMB), lambda i: (i, 0)),
            pl.BlockSpec((EMB, HID), lambda i: (0, 0)),
            pl.BlockSpec((1, HID), lambda i: (0, 0)),
        ],
        out_specs=pl.BlockSpec((KP, HID), lambda i: (i, 0)),
        out_shape=jax.ShapeDtypeStruct((K, HID), f32),
        compiler_params=pltpu.CompilerParams(
            dimension_semantics=("parallel",)),
    )(codebook, bb1_w, row(bb1_b))

    z_e = pl.pallas_call(
        _enc_body,
        grid=(NT,),
        in_specs=[
            pl.BlockSpec((MT, CONT), lambda i: (i, 0)),
            pl.BlockSpec((MT, 2), lambda i: (i, 0)),
            pl.BlockSpec((RT, 1, HID), lambda i: (i, 0, 0)),
            pl.BlockSpec((V0, CED), lambda i: (0, 0)),
            pl.BlockSpec((V1, CED), lambda i: (0, 0)),
            pl.BlockSpec((CONT, HID), lambda i: (0, 0)),
            pl.BlockSpec((1, HID), lambda i: (0, 0)),
            pl.BlockSpec((2 * CED, HID), lambda i: (0, 0)),
            pl.BlockSpec((1, HID), lambda i: (0, 0)),
            pl.BlockSpec((3 * HID, HID), lambda i: (0, 0)),
            pl.BlockSpec((1, HID), lambda i: (0, 0)),
            pl.BlockSpec((HID, EMB), lambda i: (0, 0)),
            pl.BlockSpec((1, EMB), lambda i: (0, 0)),
        ],
        out_specs=pl.BlockSpec((MT, EMB), lambda i: (i, 0)),
        out_shape=jax.ShapeDtypeStruct((N, EMB), f32),
        compiler_params=pltpu.CompilerParams(
            dimension_semantics=("parallel",)),
    )(cont.reshape(N, CONT), cat.reshape(N, 2), nf.reshape(B, 1, HID),
      emb0, emb1, cont_proj_w, row(cont_proj_b), cat_proj_w, row(cat_proj_b),
      fuse1_w, row(fuse1_b), fuse2_w, row(fuse2_b))

    z2 = (z_e ** 2).sum(1, keepdims=True)
    c2 = (codebook ** 2).sum(1).reshape(1, K)

    idx3, dloss = pl.pallas_call(
        _vq_body,
        grid=(NT,),
        in_specs=[
            pl.BlockSpec((MT, EMB), lambda i: (i, 0)),
            pl.BlockSpec((MT, 1), lambda i: (i, 0)),
            pl.BlockSpec((1, K), lambda i: (0, 0)),
            pl.BlockSpec((K, EMB), lambda i: (0, 0)),
        ],
        out_specs=[
            pl.BlockSpec((1, MT, 1), lambda i: (i, 0, 0)),
            pl.BlockSpec((1, 1, 1), lambda i: (i, 0, 0)),
        ],
        out_shape=[
            jax.ShapeDtypeStruct((NT, MT, 1), jnp.int32),
            jax.ShapeDtypeStruct((NT, 1, 1), f32),
        ],
        compiler_params=pltpu.CompilerParams(
            dimension_semantics=("parallel",)),
    )(z_e, z2, c2, codebook)

    idx_flat = idx3.reshape(N)

    g = _sc_gather(cbb, idx_flat)

    perp, vq_loss = pl.pallas_call(
        _stat_body,
        grid=(K // 128,),
        in_specs=[
            pl.BlockSpec((N, 1), lambda i: (0, 0)),
            pl.BlockSpec((NT, 1, 1), lambda i: (0, 0, 0)),
        ],
        out_specs=[
            pl.BlockSpec((1, 1), lambda i: (0, 0)),
            pl.BlockSpec((1, 1), lambda i: (0, 0)),
        ],
        out_shape=[
            jax.ShapeDtypeStruct((1, 1), f32),
            jax.ShapeDtypeStruct((1, 1), f32),
        ],
        scratch_shapes=[pltpu.SMEM((1,), f32)],
    )(idx_flat.reshape(N, 1), dloss)

    sel = (jnp.arange(MD)[None, :] ==
           (jnp.arange(MD // T) * T + (T - 1))[:, None]).astype(f32)
    cont_out, cat0, cat1, h_last = pl.pallas_call(
        _dec_body,
        grid=(ND,),
        in_specs=[
            pl.BlockSpec((MD, HID), lambda i: (i, 0)),
            pl.BlockSpec((HID, HID), lambda i: (0, 0)),
            pl.BlockSpec((1, HID), lambda i: (0, 0)),
            pl.BlockSpec((HID, CONT), lambda i: (0, 0)),
            pl.BlockSpec((1, CONT), lambda i: (0, 0)),
            pl.BlockSpec((HID, V0), lambda i: (0, 0)),
            pl.BlockSpec((1, V0), lambda i: (0, 0)),
            pl.BlockSpec((HID, V1), lambda i: (0, 0)),
            pl.BlockSpec((1, V1), lambda i: (0, 0)),
            pl.BlockSpec((MD // T, MD), lambda i: (0, 0)),
        ],
        out_specs=[
            pl.BlockSpec((MD, CONT), lambda i: (i, 0)),
            pl.BlockSpec((MD, V0), lambda i: (i, 0)),
            pl.BlockSpec((MD, V1), lambda i: (i, 0)),
            pl.BlockSpec((MD // T, HID), lambda i: (i, 0)),
        ],
        out_shape=[
            jax.ShapeDtypeStruct((N, CONT), f32),
            jax.ShapeDtypeStruct((N, V0), f32),
            jax.ShapeDtypeStruct((N, V1), f32),
            jax.ShapeDtypeStruct((B, HID), f32),
        ],
        compiler_params=pltpu.CompilerParams(
            dimension_semantics=("parallel",)),
    )(g, bb2_w, row(bb2_b), cont_head_w, row(cont_head_b),
      cat0_w, row(cat0_b), cat1_w, row(cat1_b), sel)

    canopy = pl.pallas_call(
        _can_body,
        out_shape=jax.ShapeDtypeStruct((B, 1), f32),
    )(h_last, can1_w, row(can1_b), can2_w, row(can2_b))

    return (cont_out.reshape(B, T, CONT), cat0.reshape(B, T, V0),
            cat1.reshape(B, T, V1), canopy.reshape(B),
            vq_loss.reshape(()), idx_flat.reshape(B, T),
            perp.reshape(()))


# speedup vs baseline: 2.2851x; 1.5147x over previous
import functools

import jax
import jax.numpy as jnp
from jax import lax
from jax.experimental import pallas as pl
from jax.experimental.pallas import tpu as pltpu
from jax.experimental.pallas import tpu_sc as plsc

B, T = 64, 128
N = B * T
EMB, HID, K, CONT, V0, V1, CED = 256, 128, 8192, 16, 32, 64, 6

MT = 512
NT = N // MT
RT = MT // T
KC = 2048
KP = 1024
MD = 1024
ND = N // MD
GW = 128


def _dot(a, b, dims=None):
    if dims is None:
        dims = (((1,), (0,)), ((), ()))
    return lax.dot_general(a, b, dims, precision=lax.Precision.DEFAULT,
                           preferred_element_type=jnp.float32)


def _sel(a, b, dims=None):
    if dims is None:
        dims = (((1,), (0,)), ((), ()))
    return lax.dot_general(a, b, dims, precision=lax.Precision.HIGHEST,
                           preferred_element_type=jnp.float32)


def _nf_body(naipf_ref, convwf_ref, convb_ref, npw_ref, npb_ref, o_ref):
    naipf = jnp.nan_to_num(naipf_ref[...], nan=0.0, posinf=0.0, neginf=0.0)
    feat = _dot(naipf, convwf_ref[...], (((1,), (1,)), ((), ())))
    feat = jnp.maximum(feat + convb_ref[...], 0.0)
    o_ref[...] = _dot(feat, npw_ref[...]) + npb_ref[...]


def _cbb_body(cb_ref, w_ref, b_ref, cbb_ref):
    cbb_ref[...] = _dot(cb_ref[...], w_ref[...]) + b_ref[...]


def _enc_body(cont_ref, cat_ref, nf_ref, emb0_ref, emb1_ref, cpw_ref, cpb_ref,
              kw_ref, kb_ref, f1w_ref, f1b_ref, f2w_ref, f2b_ref, z_ref):
    cont = jnp.nan_to_num(cont_ref[...], nan=0.0, posinf=0.0, neginf=0.0)
    cont_h = _dot(cont, cpw_ref[...]) + cpb_ref[...]

    cat = cat_ref[...]
    oh0 = (cat[:, 0:1] == lax.broadcasted_iota(jnp.int32, (MT, V0), 1)
           ).astype(jnp.float32)
    oh1 = (cat[:, 1:2] == lax.broadcasted_iota(jnp.int32, (MT, V1), 1)
           ).astype(jnp.float32)
    e0 = _sel(oh0, emb0_ref[...])
    e1 = _sel(oh1, emb1_ref[...])
    cat_h = _dot(jnp.concatenate([e0, e1], axis=1), kw_ref[...]) + kb_ref[...]

    na = jnp.broadcast_to(nf_ref[...], (RT, T, HID)).reshape(MT, HID)

    fused = jnp.concatenate([na, cont_h, cat_h], axis=1)
    h1 = jnp.maximum(_dot(fused, f1w_ref[...]) + f1b_ref[...], 0.0)
    z_ref[...] = _dot(h1, f2w_ref[...]) + f2b_ref[...]


def _vq_body(z_ref, z2_ref, c2_ref, cb_ref, idx_ref, dl_ref):
    z = z_ref[...]
    z2 = z2_ref[...]
    run_min = jnp.full((MT, 1), jnp.inf, jnp.float32)
    run_arg = jnp.zeros((MT, 1), jnp.int32)
    for c in range(K // KC):
        cbc = cb_ref[pl.ds(c * KC, KC), :]
        scores = _dot(z, cbc, (((1,), (1,)), ((), ())))
        d = (z2 + c2_ref[0:1, pl.ds(c * KC, KC)]) - 2.0 * scores
        cmin = jnp.min(d, axis=1, keepdims=True)
        lane = lax.broadcasted_iota(jnp.int32, (MT, KC), 1)
        carg = jnp.min(jnp.where(d == cmin, lane, K), axis=1,
                       keepdims=True) + c * KC
        upd = cmin < run_min
        run_min = jnp.where(upd, cmin, run_min)
        run_arg = jnp.where(upd, carg, run_arg)
    idx_ref[0] = run_arg
    dl_ref[...] = jnp.sum(run_min).reshape(1, 1, 1)


def _stat_body(idx_ref, dl_ref, perp_ref, loss_ref, acc_ref):
    i = pl.program_id(0)

    @pl.when(i == 0)
    def _():
        acc_ref[0] = 0.0

    ids = i * 128 + lax.broadcasted_iota(jnp.int32, (1, 128), 1)
    mask = (idx_ref[...] == ids).astype(jnp.float32)
    counts = jnp.sum(mask, axis=0, keepdims=True)
    p = counts / N
    acc_ref[0] += jnp.sum(p * jnp.log(p + 1e-12))

    @pl.when(i == K // 128 - 1)
    def _():
        perp_ref[...] = jnp.exp(-acc_ref[0]).reshape(1, 1)
        loss_ref[...] = (jnp.sum(dl_ref[...]) * (1.25 / (N * EMB))
                         ).reshape(1, 1)


def _dec_body(g_ref, b2w_ref, b2b_ref, chw_ref, chb_ref, c0w_ref, c0b_ref,
              c1w_ref, c1b_ref, sel_ref, co_ref, k0_ref, k1_ref, hl_ref):
    g = jnp.maximum(g_ref[...], 0.0)
    h = jnp.maximum(_dot(g, b2w_ref[...]) + b2b_ref[...], 0.0)
    co_ref[...] = _dot(h, chw_ref[...]) + chb_ref[...]
    k0_ref[...] = _dot(h, c0w_ref[...]) + c0b_ref[...]
    k1_ref[...] = _dot(h, c1w_ref[...]) + c1b_ref[...]
    hl_ref[...] = _sel(sel_ref[...], h)


def _can_body(hl_ref, w1_ref, b1_ref, w2_ref, b2_ref, o_ref):
    r = jnp.maximum(_dot(hl_ref[...], w1_ref[...]) + b1_ref[...], 0.0)
    o_ref[...] = _dot(r, w2_ref[...]) + b2_ref[...]


_NW = 32
_BW = N // _NW


def _sc_gather(cbb, idx_flat):
    mesh = plsc.VectorSubcoreMesh(core_axis_name="c", subcore_axis_name="s")

    @functools.partial(
        pl.kernel, mesh=mesh,
        out_type=jax.ShapeDtypeStruct((N, HID), jnp.float32),
        scratch_types=[
            pltpu.VMEM((_BW,), jnp.int32),
            pltpu.VMEM((_BW, HID), jnp.float32),
            pltpu.VMEM_SHARED((K, HID), jnp.float32),
            pltpu.SemaphoreType.DMA,
        ],
    )
    def kern(table_hbm, idx_hbm, out_hbm, idx_v, rows_v, table_sh, sem):
        @pl.when(lax.axis_index("s") == 0)
        def _():
            pltpu.sync_copy(table_hbm, table_sh)

        plsc.subcore_barrier()
        wid = lax.axis_index("s") * 2 + lax.axis_index("c")
        base = wid * _BW
        pltpu.sync_copy(idx_hbm.at[pl.ds(base, _BW)], idx_v)
        pltpu.async_copy(table_sh.at[idx_v], rows_v, sem).wait()
        pltpu.sync_copy(rows_v, out_hbm.at[pl.ds(base, _BW)])

    return kern(cbb, idx_flat)


def kernel(cont, cat, naip, emb0, emb1, conv_w, conv_b, naip_proj_w,
           naip_proj_b, cont_proj_w, cont_proj_b, cat_proj_w, cat_proj_b,
           fuse1_w, fuse1_b, fuse2_w, fuse2_b, codebook, bb1_w, bb1_b,
           bb2_w, bb2_b, cont_head_w, cont_head_b, cat0_w, cat0_b,
           cat1_w, cat1_b, can1_w, can1_b, can2_w, can2_b):
    f32 = jnp.float32
    row = lambda v: v.reshape(1, -1)

    nf = pl.pallas_call(
        _nf_body,
        out_shape=jax.ShapeDtypeStruct((B, HID), f32),
    )(naip.reshape(B, 9), conv_w.reshape(HID, 9), row(conv_b),
      naip_proj_w, row(naip_proj_b))

    cbb = pl.pallas_call(
        _cbb_body,
        grid=(K // KP,),
        in_specs=[
            pl.BlockSpec((KP, EMB), lambda i: (i, 0)),
            pl.BlockSpec((EMB, HID), lambda i: (0, 0)),
            pl.BlockSpec((1, HID), lambda i: (0, 0)),
        ],
        out_specs=pl.BlockSpec((KP, HID), lambda i: (i, 0)),
        out_shape=jax.ShapeDtypeStruct((K, HID), f32),
        compiler_params=pltpu.CompilerParams(
            dimension_semantics=("parallel",)),
    )(codebook, bb1_w, row(bb1_b))

    z_e = pl.pallas_call(
        _enc_body,
        grid=(NT,),
        in_specs=[
            pl.BlockSpec((MT, CONT), lambda i: (i, 0)),
            pl.BlockSpec((MT, 2), lambda i: (i, 0)),
            pl.BlockSpec((RT, 1, HID), lambda i: (i, 0, 0)),
            pl.BlockSpec((V0, CED), lambda i: (0, 0)),
            pl.BlockSpec((V1, CED), lambda i: (0, 0)),
            pl.BlockSpec((CONT, HID), lambda i: (0, 0)),
            pl.BlockSpec((1, HID), lambda i: (0, 0)),
            pl.BlockSpec((2 * CED, HID), lambda i: (0, 0)),
            pl.BlockSpec((1, HID), lambda i: (0, 0)),
            pl.BlockSpec((3 * HID, HID), lambda i: (0, 0)),
            pl.BlockSpec((1, HID), lambda i: (0, 0)),
            pl.BlockSpec((HID, EMB), lambda i: (0, 0)),
            pl.BlockSpec((1, EMB), lambda i: (0, 0)),
        ],
        out_specs=pl.BlockSpec((MT, EMB), lambda i: (i, 0)),
        out_shape=jax.ShapeDtypeStruct((N, EMB), f32),
        compiler_params=pltpu.CompilerParams(
            dimension_semantics=("parallel",)),
    )(cont.reshape(N, CONT), cat.reshape(N, 2), nf.reshape(B, 1, HID),
      emb0, emb1, cont_proj_w, row(cont_proj_b), cat_proj_w, row(cat_proj_b),
      fuse1_w, row(fuse1_b), fuse2_w, row(fuse2_b))

    z2 = (z_e ** 2).sum(1, keepdims=True)
    c2 = (codebook ** 2).sum(1).reshape(1, K)

    idx3, dloss = pl.pallas_call(
        _vq_body,
        grid=(NT,),
        in_specs=[
            pl.BlockSpec((MT, EMB), lambda i: (i, 0)),
            pl.BlockSpec((MT, 1), lambda i: (i, 0)),
            pl.BlockSpec((1, K), lambda i: (0, 0)),
            pl.BlockSpec((K, EMB), lambda i: (0, 0)),
        ],
        out_specs=[
            pl.BlockSpec((1, MT, 1), lambda i: (i, 0, 0)),
            pl.BlockSpec((1, 1, 1), lambda i: (i, 0, 0)),
        ],
        out_shape=[
            jax.ShapeDtypeStruct((NT, MT, 1), jnp.int32),
            jax.ShapeDtypeStruct((NT, 1, 1), f32),
        ],
        compiler_params=pltpu.CompilerParams(
            dimension_semantics=("parallel",)),
    )(z_e, z2, c2, codebook)

    idx_flat = idx3.reshape(N)

    g = _sc_gather(cbb, idx_flat)

    perp, vq_loss = pl.pallas_call(
        _stat_body,
        grid=(K // 128,),
        in_specs=[
            pl.BlockSpec((N, 1), lambda i: (0, 0)),
            pl.BlockSpec((NT, 1, 1), lambda i: (0, 0, 0)),
        ],
        out_specs=[
            pl.BlockSpec((1, 1), lambda i: (0, 0)),
            pl.BlockSpec((1, 1), lambda i: (0, 0)),
        ],
        out_shape=[
            jax.ShapeDtypeStruct((1, 1), f32),
            jax.ShapeDtypeStruct((1, 1), f32),
        ],
        scratch_shapes=[pltpu.SMEM((1,), f32)],
    )(idx_flat.reshape(N, 1), dloss)

    sel = (jnp.arange(MD)[None, :] ==
           (jnp.arange(MD // T) * T + (T - 1))[:, None]).astype(f32)
    cont_out, cat0, cat1, h_last = pl.pallas_call(
        _dec_body,
        grid=(ND,),
        in_specs=[
            pl.BlockSpec((MD, HID), lambda i: (i, 0)),
            pl.BlockSpec((HID, HID), lambda i: (0, 0)),
            pl.BlockSpec((1, HID), lambda i: (0, 0)),
            pl.BlockSpec((HID, CONT), lambda i: (0, 0)),
            pl.BlockSpec((1, CONT), lambda i: (0, 0)),
            pl.BlockSpec((HID, V0), lambda i: (0, 0)),
            pl.BlockSpec((1, V0), lambda i: (0, 0)),
            pl.BlockSpec((HID, V1), lambda i: (0, 0)),
            pl.BlockSpec((1, V1), lambda i: (0, 0)),
            pl.BlockSpec((MD // T, MD), lambda i: (0, 0)),
        ],
        out_specs=[
            pl.BlockSpec((MD, CONT), lambda i: (i, 0)),
            pl.BlockSpec((MD, V0), lambda i: (i, 0)),
            pl.BlockSpec((MD, V1), lambda i: (i, 0)),
            pl.BlockSpec((MD // T, HID), lambda i: (i, 0)),
        ],
        out_shape=[
            jax.ShapeDtypeStruct((N, CONT), f32),
            jax.ShapeDtypeStruct((N, V0), f32),
            jax.ShapeDtypeStruct((N, V1), f32),
            jax.ShapeDtypeStruct((B, HID), f32),
        ],
        compiler_params=pltpu.CompilerParams(
            dimension_semantics=("parallel",)),
    )(g, bb2_w, row(bb2_b), cont_head_w, row(cont_head_b),
      cat0_w, row(cat0_b), cat1_w, row(cat1_b), sel)

    canopy = pl.pallas_call(
        _can_body,
        out_shape=jax.ShapeDtypeStruct((B, 1), f32),
    )(h_last, can1_w, row(can1_b), can2_w, row(can2_b))

    return (cont_out.reshape(B, T, CONT), cat0.reshape(B, T, V0),
            cat1.reshape(B, T, V1), canopy.reshape(B),
            vq_loss.reshape(()), idx_flat.reshape(B, T),
            perp.reshape(()))


# speedup vs baseline: 3.1109x; 1.3614x over previous
import functools

import jax
import jax.numpy as jnp
from jax import lax
from jax.experimental import pallas as pl
from jax.experimental.pallas import tpu as pltpu
from jax.experimental.pallas import tpu_sc as plsc

B, T = 64, 128
N = B * T
EMB, HID, K, CONT, V0, V1, CED = 256, 128, 8192, 16, 32, 64, 6

MT = 512
NT = N // MT
RT = MT // T
KC = 2048
KP = 1024
MD = 1024
ND = N // MD
GW = 128


def _dot(a, b, dims=None):
    if dims is None:
        dims = (((1,), (0,)), ((), ()))
    return lax.dot_general(a, b, dims, precision=lax.Precision.DEFAULT,
                           preferred_element_type=jnp.float32)


def _sel(a, b, dims=None):
    if dims is None:
        dims = (((1,), (0,)), ((), ()))
    return lax.dot_general(a, b, dims, precision=lax.Precision.HIGHEST,
                           preferred_element_type=jnp.float32)


def _nf_body(naipf_ref, convwf_ref, convb_ref, npw_ref, npb_ref, o_ref):
    naipf = jnp.nan_to_num(naipf_ref[...], nan=0.0, posinf=0.0, neginf=0.0)
    feat = _dot(naipf, convwf_ref[...], (((1,), (1,)), ((), ())))
    feat = jnp.maximum(feat + convb_ref[...], 0.0)
    o_ref[...] = _dot(feat, npw_ref[...]) + npb_ref[...]


def _cbb_body(cb_ref, w_ref, b_ref, cbb_ref):
    cbb_ref[...] = _dot(cb_ref[...], w_ref[...]) + b_ref[...]


def _enc_body(cont_ref, cat_ref, nf_ref, emb0_ref, emb1_ref, cpw_ref, cpb_ref,
              kw_ref, kb_ref, f1w_ref, f1b_ref, f2w_ref, f2b_ref, z_ref):
    cont = jnp.nan_to_num(cont_ref[...], nan=0.0, posinf=0.0, neginf=0.0)
    cont_h = _dot(cont, cpw_ref[...]) + cpb_ref[...]

    cat = cat_ref[...]
    oh0 = (cat[:, 0:1] == lax.broadcasted_iota(jnp.int32, (MT, V0), 1)
           ).astype(jnp.float32)
    oh1 = (cat[:, 1:2] == lax.broadcasted_iota(jnp.int32, (MT, V1), 1)
           ).astype(jnp.float32)
    e0 = _sel(oh0, emb0_ref[...])
    e1 = _sel(oh1, emb1_ref[...])
    cat_h = _dot(jnp.concatenate([e0, e1], axis=1), kw_ref[...]) + kb_ref[...]

    na = jnp.broadcast_to(nf_ref[...], (RT, T, HID)).reshape(MT, HID)

    fused = jnp.concatenate([na, cont_h, cat_h], axis=1)
    h1 = jnp.maximum(_dot(fused, f1w_ref[...]) + f1b_ref[...], 0.0)
    z_ref[...] = _dot(h1, f2w_ref[...]) + f2b_ref[...]


def _vq_body(z_ref, z2_ref, c2_ref, cb_ref, idx_ref, dl_ref):
    z = z_ref[...]
    z2 = z2_ref[...]
    run_min = jnp.full((MT, 1), jnp.inf, jnp.float32)
    run_arg = jnp.zeros((MT, 1), jnp.int32)
    for c in range(K // KC):
        cbc = cb_ref[pl.ds(c * KC, KC), :]
        scores = _dot(z, cbc, (((1,), (1,)), ((), ())))
        d = (z2 + c2_ref[0:1, pl.ds(c * KC, KC)]) - 2.0 * scores
        cmin = jnp.min(d, axis=1, keepdims=True)
        lane = lax.broadcasted_iota(jnp.int32, (MT, KC), 1)
        carg = jnp.min(jnp.where(d == cmin, lane, K), axis=1,
                       keepdims=True) + c * KC
        upd = cmin < run_min
        run_min = jnp.where(upd, cmin, run_min)
        run_arg = jnp.where(upd, carg, run_arg)
    idx_ref[0] = run_arg
    dl_ref[...] = jnp.sum(run_min).reshape(1, 1, 1)


def _stat_body(idx_ref, dl_ref, perp_ref, loss_ref):
    idx = idx_ref[...]
    hi = idx // 128
    lo = idx - hi * 128
    hi_oh = (hi == lax.broadcasted_iota(jnp.int32, (N, K // 128), 1)
             ).astype(jnp.float32)
    lo_oh = (lo == lax.broadcasted_iota(jnp.int32, (N, 128), 1)
             ).astype(jnp.float32)
    counts = _dot(hi_oh, lo_oh, (((0,), (0,)), ((), ())))
    p = counts / N
    ent = jnp.sum(p * jnp.log(p + 1e-12))
    perp_ref[...] = jnp.exp(-ent).reshape(1, 1)
    loss_ref[...] = (jnp.sum(dl_ref[...]) * (1.25 / (N * EMB))).reshape(1, 1)


def _dec_body(g_ref, b2w_ref, b2b_ref, chw_ref, chb_ref, c0w_ref, c0b_ref,
              c1w_ref, c1b_ref, sel_ref, co_ref, k0_ref, k1_ref, hl_ref):
    g = jnp.maximum(g_ref[...], 0.0)
    h = jnp.maximum(_dot(g, b2w_ref[...]) + b2b_ref[...], 0.0)
    co_ref[...] = _dot(h, chw_ref[...]) + chb_ref[...]
    k0_ref[...] = _dot(h, c0w_ref[...]) + c0b_ref[...]
    k1_ref[...] = _dot(h, c1w_ref[...]) + c1b_ref[...]
    hl_ref[...] = _sel(sel_ref[...], h)


def _can_body(hl_ref, w1_ref, b1_ref, w2_ref, b2_ref, o_ref):
    r = jnp.maximum(_dot(hl_ref[...], w1_ref[...]) + b1_ref[...], 0.0)
    o_ref[...] = _dot(r, w2_ref[...]) + b2_ref[...]


_NW = 32
_BW = N // _NW


def _sc_gather(cbb, idx_flat):
    mesh = plsc.VectorSubcoreMesh(core_axis_name="c", subcore_axis_name="s")

    @functools.partial(
        pl.kernel, mesh=mesh,
        out_type=jax.ShapeDtypeStruct((N, HID), jnp.float32),
        scratch_types=[
            pltpu.VMEM((_BW,), jnp.int32),
            pltpu.VMEM((_BW, HID), jnp.float32),
            pltpu.VMEM_SHARED((K, HID), jnp.float32),
            pltpu.SemaphoreType.DMA,
        ],
    )
    def kern(table_hbm, idx_hbm, out_hbm, idx_v, rows_v, table_sh, sem):
        @pl.when(lax.axis_index("s") == 0)
        def _():
            pltpu.sync_copy(table_hbm, table_sh)

        plsc.subcore_barrier()
        wid = lax.axis_index("s") * 2 + lax.axis_index("c")
        base = wid * _BW
        pltpu.sync_copy(idx_hbm.at[pl.ds(base, _BW)], idx_v)
        pltpu.async_copy(table_sh.at[idx_v], rows_v, sem).wait()
        pltpu.sync_copy(rows_v, out_hbm.at[pl.ds(base, _BW)])

    return kern(cbb, idx_flat)


def kernel(cont, cat, naip, emb0, emb1, conv_w, conv_b, naip_proj_w,
           naip_proj_b, cont_proj_w, cont_proj_b, cat_proj_w, cat_proj_b,
           fuse1_w, fuse1_b, fuse2_w, fuse2_b, codebook, bb1_w, bb1_b,
           bb2_w, bb2_b, cont_head_w, cont_head_b, cat0_w, cat0_b,
           cat1_w, cat1_b, can1_w, can1_b, can2_w, can2_b):
    f32 = jnp.float32
    row = lambda v: v.reshape(1, -1)

    nf = pl.pallas_call(
        _nf_body,
        out_shape=jax.ShapeDtypeStruct((B, HID), f32),
    )(naip.reshape(B, 9), conv_w.reshape(HID, 9), row(conv_b),
      naip_proj_w, row(naip_proj_b))

    cbb = pl.pallas_call(
        _cbb_body,
        grid=(K // KP,),
        in_specs=[
            pl.BlockSpec((KP, EMB), lambda i: (i, 0)),
            pl.BlockSpec((EMB, HID), lambda i: (0, 0)),
            pl.BlockSpec((1, HID), lambda i: (0, 0)),
        ],
        out_specs=pl.BlockSpec((KP, HID), lambda i: (i, 0)),
        out_shape=jax.ShapeDtypeStruct((K, HID), f32),
        compiler_params=pltpu.CompilerParams(
            dimension_semantics=("parallel",)),
    )(codebook, bb1_w, row(bb1_b))

    z_e = pl.pallas_call(
        _enc_body,
        grid=(NT,),
        in_specs=[
            pl.BlockSpec((MT, CONT), lambda i: (i, 0)),
            pl.BlockSpec((MT, 2), lambda i: (i, 0)),
            pl.BlockSpec((RT, 1, HID), lambda i: (i, 0, 0)),
            pl.BlockSpec((V0, CED), lambda i: (0, 0)),
            pl.BlockSpec((V1, CED), lambda i: (0, 0)),
            pl.BlockSpec((CONT, HID), lambda i: (0, 0)),
            pl.BlockSpec((1, HID), lambda i: (0, 0)),
            pl.BlockSpec((2 * CED, HID), lambda i: (0, 0)),
            pl.BlockSpec((1, HID), lambda i: (0, 0)),
            pl.BlockSpec((3 * HID, HID), lambda i: (0, 0)),
            pl.BlockSpec((1, HID), lambda i: (0, 0)),
            pl.BlockSpec((HID, EMB), lambda i: (0, 0)),
            pl.BlockSpec((1, EMB), lambda i: (0, 0)),
        ],
        out_specs=pl.BlockSpec((MT, EMB), lambda i: (i, 0)),
        out_shape=jax.ShapeDtypeStruct((N, EMB), f32),
        compiler_params=pltpu.CompilerParams(
            dimension_semantics=("parallel",)),
    )(cont.reshape(N, CONT), cat.reshape(N, 2), nf.reshape(B, 1, HID),
      emb0, emb1, cont_proj_w, row(cont_proj_b), cat_proj_w, row(cat_proj_b),
      fuse1_w, row(fuse1_b), fuse2_w, row(fuse2_b))

    z2 = (z_e ** 2).sum(1, keepdims=True)
    c2 = (codebook ** 2).sum(1).reshape(1, K)

    idx3, dloss = pl.pallas_call(
        _vq_body,
        grid=(NT,),
        in_specs=[
            pl.BlockSpec((MT, EMB), lambda i: (i, 0)),
            pl.BlockSpec((MT, 1), lambda i: (i, 0)),
            pl.BlockSpec((1, K), lambda i: (0, 0)),
            pl.BlockSpec((K, EMB), lambda i: (0, 0)),
        ],
        out_specs=[
            pl.BlockSpec((1, MT, 1), lambda i: (i, 0, 0)),
            pl.BlockSpec((1, 1, 1), lambda i: (i, 0, 0)),
        ],
        out_shape=[
            jax.ShapeDtypeStruct((NT, MT, 1), jnp.int32),
            jax.ShapeDtypeStruct((NT, 1, 1), f32),
        ],
        compiler_params=pltpu.CompilerParams(
            dimension_semantics=("parallel",)),
    )(z_e, z2, c2, codebook)

    idx_flat = idx3.reshape(N)

    g = _sc_gather(cbb, idx_flat)

    perp, vq_loss = pl.pallas_call(
        _stat_body,
        out_shape=[
            jax.ShapeDtypeStruct((1, 1), f32),
            jax.ShapeDtypeStruct((1, 1), f32),
        ],
    )(idx_flat.reshape(N, 1), dloss)

    sel = (jnp.arange(MD)[None, :] ==
           (jnp.arange(MD // T) * T + (T - 1))[:, None]).astype(f32)
    cont_out, cat0, cat1, h_last = pl.pallas_call(
        _dec_body,
        grid=(ND,),
        in_specs=[
            pl.BlockSpec((MD, HID), lambda i: (i, 0)),
            pl.BlockSpec((HID, HID), lambda i: (0, 0)),
            pl.BlockSpec((1, HID), lambda i: (0, 0)),
            pl.BlockSpec((HID, CONT), lambda i: (0, 0)),
            pl.BlockSpec((1, CONT), lambda i: (0, 0)),
            pl.BlockSpec((HID, V0), lambda i: (0, 0)),
            pl.BlockSpec((1, V0), lambda i: (0, 0)),
            pl.BlockSpec((HID, V1), lambda i: (0, 0)),
            pl.BlockSpec((1, V1), lambda i: (0, 0)),
            pl.BlockSpec((MD // T, MD), lambda i: (0, 0)),
        ],
        out_specs=[
            pl.BlockSpec((MD, CONT), lambda i: (i, 0)),
            pl.BlockSpec((MD, V0), lambda i: (i, 0)),
            pl.BlockSpec((MD, V1), lambda i: (i, 0)),
            pl.BlockSpec((MD // T, HID), lambda i: (i, 0)),
        ],
        out_shape=[
            jax.ShapeDtypeStruct((N, CONT), f32),
            jax.ShapeDtypeStruct((N, V0), f32),
            jax.ShapeDtypeStruct((N, V1), f32),
            jax.ShapeDtypeStruct((B, HID), f32),
        ],
        compiler_params=pltpu.CompilerParams(
            dimension_semantics=("parallel",)),
    )(g, bb2_w, row(bb2_b), cont_head_w, row(cont_head_b),
      cat0_w, row(cat0_b), cat1_w, row(cat1_b), sel)

    canopy = pl.pallas_call(
        _can_body,
        out_shape=jax.ShapeDtypeStruct((B, 1), f32),
    )(h_last, can1_w, row(can1_b), can2_w, row(can2_b))

    return (cont_out.reshape(B, T, CONT), cat0.reshape(B, T, V0),
            cat1.reshape(B, T, V1), canopy.reshape(B),
            vq_loss.reshape(()), idx_flat.reshape(B, T),
            perp.reshape(()))
